# baseline (speedup 1.0000x reference)
import functools

import jax
import jax.numpy as jnp
import numpy as np
from jax import lax
from jax.experimental import pallas as pl
from jax.experimental.pallas import tpu as pltpu
from jax.experimental.pallas import tpu_sc as plsc

NC = 2
NS = 16
L = 16
NW = NC * NS

VOCAB = 100000
DIM = 64
BATCH = 4096
HIST = 50
B_TOTAL = BATCH * HIST
ROWS_PER_W = B_TOTAL // NW
BBLK = BATCH // NW
NB = 5
NWAVE = HIST // NB


def _xlane_gather(v, idx):
    dnums = lax.GatherDimensionNumbers(
        offset_dims=(), collapsed_slice_dims=(0,), start_index_map=(0,))
    return lax.gather(
        v, idx[:, None], dnums, (1,),
        mode=lax.GatherScatterMode.PROMISE_IN_BOUNDS)


def _body(x_hbm, tab_hbm, conc_hbm, out_hbm, idxraw, idxb, idx2, hitsv,
          conc_v, *rest):
    bufs = rest[:NB]
    tbufs = rest[NB:2 * NB]
    sems = rest[2 * NB:]
    lanes = lax.iota(jnp.int32, L)
    kvecs = [lanes + k * L for k in range(BBLK // L)]

    wid = lax.axis_index("s") * NC + lax.axis_index("c")
    bb0 = wid * BBLK

    pltpu.sync_copy(x_hbm.at[pl.ds(wid * ROWS_PER_W, ROWS_PER_W)], idxraw)
    pltpu.sync_copy(conc_hbm, conc_v)

    cvecs = [conc_v[pl.ds(q * L, L)] for q in range(DIM // L)]

    def repack_chunk(h, carry):
        acc = jnp.zeros((L,), jnp.int32)
        for k in range(BBLK // L):
            v = plsc.load_gather(idxraw, [kvecs[k] * HIST + h])
            m = v == VOCAB
            idxb[h, pl.ds(k * L, L)] = v
            idx2[h, pl.ds(k * L, L)] = jnp.where(m, 0, v)
            acc = acc | jnp.where(m, 1, 0)
        for d in (8, 4, 2, 1):
            acc = acc | _xlane_gather(acc, lanes ^ d)
        hitsv[h, pl.ds(0, L)] = acc
        return carry

    lax.fori_loop(0, HIST, repack_chunk, 0)

    def transpose_chunk(buf, tbuf):
        def tcol(d4, carry):
            for dd in range(4):
                d = d4 * 4 + dd
                dv = jnp.zeros((L,), jnp.int32) + d
                for k in range(BBLK // L):
                    vv = plsc.load_gather(buf, [kvecs[k], dv])
                    plsc.store_scatter(tbuf, [dv, kvecs[k]], vv)
            return carry

        lax.fori_loop(0, DIM // 4, tcol, 0)

    def patch_chunk(h, tbuf):
        hv = jnp.zeros((L,), jnp.int32) + h

        def groups(j, carry):
            base = j * L
            v = plsc.load_gather(idxb, [hv, lanes + base])
            for r in range(L):
                @pl.when(v[r] == VOCAB)
                def _fix(r=r):
                    col = jnp.zeros((L,), jnp.int32) + (base + r)
                    for q in range(DIM // L):
                        plsc.store_scatter(
                            tbuf, [lanes + q * L, col], cvecs[q])
            return carry

        lax.fori_loop(0, BBLK // L, groups, 0)

    def wave(w, carry):
        handles = []
        for b in range(NB):
            h = w * NB + b

            @pl.when(w > 0)
            def _drain(b=b, h=h):
                pltpu.make_async_copy(
                    tbufs[b], out_hbm.at[h, :, pl.ds(bb0, BBLK)],
                    sems[b]).wait()

            handles.append(pltpu.async_copy(
                tab_hbm.at[idx2.at[h]], bufs[b], sems[b]))

        for b in range(NB):
            h = w * NB + b
            handles[b].wait()
            transpose_chunk(bufs[b], tbufs[b])

            hv = hitsv[h, pl.ds(0, L)]

            @pl.when(hv[0] > 0)
            def _patch(b=b, h=h):
                patch_chunk(h, tbufs[b])

            pltpu.async_copy(
                tbufs[b], out_hbm.at[h, :, pl.ds(bb0, BBLK)], sems[b])
        return carry

    lax.fori_loop(0, NWAVE, wave, 0)

    for b in range(NB):
        h = (NWAVE - 1) * NB + b
        pltpu.make_async_copy(
            tbufs[b], out_hbm.at[h, :, pl.ds(bb0, BBLK)], sems[b]).wait()


@jax.jit
def _lookup(x1d, table, conc1d):
    scratch = [
        pltpu.VMEM((ROWS_PER_W,), jnp.int32),
        pltpu.VMEM((HIST, BBLK), jnp.int32),
        pltpu.VMEM((HIST, BBLK), jnp.int32),
        pltpu.VMEM((HIST, L), jnp.int32),
        pltpu.VMEM((DIM,), jnp.float32),
    ]
    scratch += [pltpu.VMEM((BBLK, DIM), jnp.float32) for _ in range(NB)]
    scratch += [pltpu.VMEM((DIM, BBLK), jnp.float32) for _ in range(NB)]
    scratch += [pltpu.SemaphoreType.DMA for _ in range(NB)]
    run = pl.kernel(
        _body,
        out_type=jax.ShapeDtypeStruct((HIST, DIM, BATCH), jnp.float32),
        mesh=plsc.VectorSubcoreMesh(core_axis_name="c", subcore_axis_name="s"),
        scratch_types=scratch,
        compiler_params=pltpu.CompilerParams(
            use_tc_tiling_on_sc=False, needs_layout_passes=False),
    )
    return run(x1d, table, conc1d)


def kernel(x, embed_weight, concepts):
    x1d = x.reshape(B_TOTAL).astype(jnp.int32)
    out3 = _lookup(x1d, embed_weight, concepts.reshape(DIM))
    return jnp.transpose(out3, (2, 0, 1))


# speedup vs baseline: 2.6832x; 2.6832x over previous
import functools

import jax
import jax.numpy as jnp
import numpy as np
from jax import lax
from jax.experimental import pallas as pl
from jax.experimental.pallas import tpu as pltpu
from jax.experimental.pallas import tpu_sc as plsc

NC = 2
NS = 16
L = 16
NW = NC * NS

VOCAB = 100000
DIM = 64
BATCH = 4096
HIST = 50
B_TOTAL = BATCH * HIST
ROWS_PER_W = B_TOTAL // NW
BBLK = BATCH // NW
NB = 5
NWAVE = HIST // NB


def _xlane_gather(v, idx):
    dnums = lax.GatherDimensionNumbers(
        offset_dims=(), collapsed_slice_dims=(0,), start_index_map=(0,))
    return lax.gather(
        v, idx[:, None], dnums, (1,),
        mode=lax.GatherScatterMode.PROMISE_IN_BOUNDS)


def _body(x_hbm, tab_hbm, conc_hbm, out_hbm, idxraw, idxb, idx2, hitsv,
          conc_v, *rest):
    bufs = rest[:NB]
    tbufs = rest[NB:2 * NB]
    sems = rest[2 * NB:]
    lanes = lax.iota(jnp.int32, L)
    kvecs = [lanes + k * L for k in range(BBLK // L)]

    wid = lax.axis_index("s") * NC + lax.axis_index("c")
    bb0 = wid * BBLK

    pltpu.sync_copy(x_hbm.at[pl.ds(wid * ROWS_PER_W, ROWS_PER_W)], idxraw)
    pltpu.sync_copy(conc_hbm, conc_v)

    cvecs = [conc_v[pl.ds(q * L, L)] for q in range(DIM // L)]

    def repack_chunk(h, carry):
        acc = jnp.zeros((L,), jnp.int32)
        for k in range(BBLK // L):
            v = plsc.load_gather(idxraw, [kvecs[k] * HIST + h])
            m = v == VOCAB
            idxb[h, pl.ds(k * L, L)] = v
            idx2[h, pl.ds(k * L, L)] = jnp.where(m, 0, v)
            acc = acc | jnp.where(m, 1, 0)
        for d in (8, 4, 2, 1):
            acc = acc | _xlane_gather(acc, lanes ^ d)
        hitsv[h, pl.ds(0, L)] = acc
        return carry

    lax.fori_loop(0, HIST, repack_chunk, 0)

    def transpose_chunk(buf, tbuf):
        @functools.partial(plsc.parallel_loop, 0, DIM, unroll=4)
        def _t(d):
            dv = jnp.zeros((L,), jnp.int32) + d
            for k in range(BBLK // L):
                vv = plsc.load_gather(buf, [kvecs[k], dv])
                plsc.store_scatter(tbuf, [dv, kvecs[k]], vv)

    def patch_chunk(h, tbuf):
        hv = jnp.zeros((L,), jnp.int32) + h

        def groups(j, carry):
            base = j * L
            v = plsc.load_gather(idxb, [hv, lanes + base])
            for r in range(L):
                @pl.when(v[r] == VOCAB)
                def _fix(r=r):
                    col = jnp.zeros((L,), jnp.int32) + (base + r)
                    for q in range(DIM // L):
                        plsc.store_scatter(
                            tbuf, [lanes + q * L, col], cvecs[q])
            return carry

        lax.fori_loop(0, BBLK // L, groups, 0)

    def wave(w, carry):
        handles = []
        for b in range(NB):
            h = w * NB + b

            @pl.when(w > 0)
            def _drain(b=b, h=h):
                pltpu.make_async_copy(
                    tbufs[b], out_hbm.at[h, :, pl.ds(bb0, BBLK)],
                    sems[b]).wait()

            handles.append(pltpu.async_copy(
                tab_hbm.at[idx2.at[h]], bufs[b], sems[b]))

        for b in range(NB):
            h = w * NB + b
            handles[b].wait()
            transpose_chunk(bufs[b], tbufs[b])

            hv = hitsv[h, pl.ds(0, L)]

            @pl.when(hv[0] > 0)
            def _patch(b=b, h=h):
                patch_chunk(h, tbufs[b])

            pltpu.async_copy(
                tbufs[b], out_hbm.at[h, :, pl.ds(bb0, BBLK)], sems[b])
        return carry

    lax.fori_loop(0, NWAVE, wave, 0)

    for b in range(NB):
        h = (NWAVE - 1) * NB + b
        pltpu.make_async_copy(
            tbufs[b], out_hbm.at[h, :, pl.ds(bb0, BBLK)], sems[b]).wait()


@jax.jit
def _lookup(x1d, table, conc1d):
    scratch = [
        pltpu.VMEM((ROWS_PER_W,), jnp.int32),
        pltpu.VMEM((HIST, BBLK), jnp.int32),
        pltpu.VMEM((HIST, BBLK), jnp.int32),
        pltpu.VMEM((HIST, L), jnp.int32),
        pltpu.VMEM((DIM,), jnp.float32),
    ]
    scratch += [pltpu.VMEM((BBLK, DIM), jnp.float32) for _ in range(NB)]
    scratch += [pltpu.VMEM((DIM, BBLK), jnp.float32) for _ in range(NB)]
    scratch += [pltpu.SemaphoreType.DMA for _ in range(NB)]
    run = pl.kernel(
        _body,
        out_type=jax.ShapeDtypeStruct((HIST, DIM, BATCH), jnp.float32),
        mesh=plsc.VectorSubcoreMesh(core_axis_name="c", subcore_axis_name="s"),
        scratch_types=scratch,
        compiler_params=pltpu.CompilerParams(
            use_tc_tiling_on_sc=False, needs_layout_passes=False),
    )
    return run(x1d, table, conc1d)


def kernel(x, embed_weight, concepts):
    x1d = x.reshape(B_TOTAL).astype(jnp.int32)
    out3 = _lookup(x1d, embed_weight, concepts.reshape(DIM))
    return jnp.transpose(out3, (2, 0, 1))


# speedup vs baseline: 3.8235x; 1.4250x over previous
import functools

import jax
import jax.numpy as jnp
import numpy as np
from jax import lax
from jax.experimental import pallas as pl
from jax.experimental.pallas import tpu as pltpu
from jax.experimental.pallas import tpu_sc as plsc

NC = 2
NS = 16
L = 16
NW = NC * NS

VOCAB = 100000
DIM = 64
BATCH = 4096
HIST = 50
B_TOTAL = BATCH * HIST
ROWS_PER_W = B_TOTAL // NW
BBLK = BATCH // NW
NB = 5
NWAVE = HIST // NB


def _xlane_gather(v, idx):
    dnums = lax.GatherDimensionNumbers(
        offset_dims=(), collapsed_slice_dims=(0,), start_index_map=(0,))
    return lax.gather(
        v, idx[:, None], dnums, (1,),
        mode=lax.GatherScatterMode.PROMISE_IN_BOUNDS)


def _body(x_hbm, tab_hbm, conc_hbm, out_hbm, idxraw, idxb, idx2, hitsv,
          conc_v, *rest):
    bufs = rest[:NB]
    tbufs = rest[NB:2 * NB]
    sems = rest[2 * NB:]
    lanes = lax.iota(jnp.int32, L)
    kvecs = [lanes + k * L for k in range(BBLK // L)]

    wid = lax.axis_index("s") * NC + lax.axis_index("c")
    bb0 = wid * BBLK

    pltpu.sync_copy(x_hbm.at[pl.ds(wid * ROWS_PER_W, ROWS_PER_W)], idxraw)
    pltpu.sync_copy(conc_hbm, conc_v)

    cvecs = [conc_v[pl.ds(q * L, L)] for q in range(DIM // L)]

    def repack_chunk(h, carry):
        acc = jnp.zeros((L,), jnp.int32)
        for k in range(BBLK // L):
            v = plsc.load_gather(idxraw, [kvecs[k] * HIST + h])
            m = v == VOCAB
            idxb[h, pl.ds(k * L, L)] = v
            idx2[h, pl.ds(k * L, L)] = jnp.where(m, 0, v)
            acc = acc | jnp.where(m, 1, 0)
        for d in (8, 4, 2, 1):
            acc = acc | _xlane_gather(acc, lanes ^ d)
        hitsv[h, pl.ds(0, L)] = acc
        return carry

    lax.fori_loop(0, HIST, repack_chunk, 0)

    def transpose_chunk(buf, tbuf):
        @functools.partial(plsc.parallel_loop, 0, DIM, unroll=4)
        def _t(d):
            dv = jnp.zeros((L,), jnp.int32) + d
            for k in range(BBLK // L):
                vv = plsc.load_gather(buf, [kvecs[k], dv])
                plsc.store_scatter(tbuf, [dv, kvecs[k]], vv)

    def patch_chunk(h, tbuf):
        hv = jnp.zeros((L,), jnp.int32) + h

        def groups(j, carry):
            base = j * L
            v = plsc.load_gather(idxb, [hv, lanes + base])
            for r in range(L):
                @pl.when(v[r] == VOCAB)
                def _fix(r=r):
                    col = jnp.zeros((L,), jnp.int32) + (base + r)
                    for q in range(DIM // L):
                        plsc.store_scatter(
                            tbuf, [lanes + q * L, col], cvecs[q])
            return carry

        lax.fori_loop(0, BBLK // L, groups, 0)

    def wave(w, carry):
        handles = []
        for b in range(NB):
            h = w * NB + b

            @pl.when(w > 0)
            def _drain(b=b, h=h):
                for ti in range(DIM // 8):
                    pltpu.make_async_copy(
                        tbufs[b].at[pl.ds(ti * 8, 8), :],
                        out_hbm.at[h, ti, wid], sems[b]).wait()

            handles.append(pltpu.async_copy(
                tab_hbm.at[idx2.at[h]], bufs[b], sems[b]))

        for b in range(NB):
            h = w * NB + b
            handles[b].wait()
            transpose_chunk(bufs[b], tbufs[b])

            hv = hitsv[h, pl.ds(0, L)]

            @pl.when(hv[0] > 0)
            def _patch(b=b, h=h):
                patch_chunk(h, tbufs[b])

            for ti in range(DIM // 8):
                pltpu.async_copy(
                    tbufs[b].at[pl.ds(ti * 8, 8), :],
                    out_hbm.at[h, ti, wid], sems[b])
        return carry

    lax.fori_loop(0, NWAVE, wave, 0)

    for b in range(NB):
        h = (NWAVE - 1) * NB + b
        for ti in range(DIM // 8):
            pltpu.make_async_copy(
                tbufs[b].at[pl.ds(ti * 8, 8), :],
                out_hbm.at[h, ti, wid], sems[b]).wait()


@jax.jit
def _lookup(x1d, table, conc1d):
    scratch = [
        pltpu.VMEM((ROWS_PER_W,), jnp.int32),
        pltpu.VMEM((HIST, BBLK), jnp.int32),
        pltpu.VMEM((HIST, BBLK), jnp.int32),
        pltpu.VMEM((HIST, L), jnp.int32),
        pltpu.VMEM((DIM,), jnp.float32),
    ]
    scratch += [pltpu.VMEM((BBLK, DIM), jnp.float32) for _ in range(NB)]
    scratch += [pltpu.VMEM((DIM, BBLK), jnp.float32) for _ in range(NB)]
    scratch += [pltpu.SemaphoreType.DMA for _ in range(NB)]
    run = pl.kernel(
        _body,
        out_type=jax.ShapeDtypeStruct((HIST, DIM // 8, NW, 8, BBLK),
                                      jnp.float32),
        mesh=plsc.VectorSubcoreMesh(core_axis_name="c", subcore_axis_name="s"),
        scratch_types=scratch,
        compiler_params=pltpu.CompilerParams(
            use_tc_tiling_on_sc=False, needs_layout_passes=False),
    )
    return run(x1d, table, conc1d)


def kernel(x, embed_weight, concepts):
    x1d = x.reshape(B_TOTAL).astype(jnp.int32)
    out5 = _lookup(x1d, embed_weight, concepts.reshape(DIM))
    return out5.transpose(2, 4, 0, 1, 3).reshape(BATCH, HIST, DIM)
